# Initial kernel scaffold; baseline (speedup 1.0000x reference)
import functools

import jax
import jax.numpy as jnp
from jax import lax
from jax.experimental import pallas as pl
from jax.experimental.pallas import tpu as pltpu
from jax.experimental.pallas import tpu_sc as plsc

N = 10000
D = 256
H = 256
E = 160000
HALF = 128

NS = 16
EPT = E // NS
B = 80
NCH = EPT // B
RPT = N // NS
MM_R = 1000


def _mm_body(x_ref, w_ref, lna, lnb, c1a, c1b, c2a, c2b):
    y = jnp.dot(x_ref[...], w_ref[...], preferred_element_type=jnp.float32)
    lna[...] = y[:, 0:128]
    lnb[...] = y[:, 128:256]
    c1a[...] = y[:, 256:384]
    c1b[...] = y[:, 384:512]
    c2a[...] = y[:, 512:640]
    c2b[...] = y[:, 640:768]


def _matmul3(x, wcat):
    return pl.pallas_call(
        _mm_body,
        grid=(N // MM_R,),
        in_specs=[
            pl.BlockSpec((MM_R, D), lambda i: (i, 0)),
            pl.BlockSpec((D, 3 * H), lambda i: (0, 0)),
        ],
        out_specs=[pl.BlockSpec((MM_R, HALF), lambda i: (i, 0))] * 6,
        out_shape=[jax.ShapeDtypeStruct((N, HALF), jnp.float32)] * 6,
    )(x, wcat)


def _sc_body(lna, lnb, t1a, t1b, t2a, t2b,
             s1, d1, e1, s2, d2, e2,
             out, acc, srcv, dstv, eav, rows, sem):
    c = lax.axis_index("c")
    s = lax.axis_index("s")
    r0 = s * RPT

    def half(ln, t1, t2, col0):
        pltpu.sync_copy(ln.at[pl.ds(r0, RPT)], acc.at[pl.ds(r0, RPT)])
        plsc.subcore_barrier()

        for (t, sv, dv, ev) in ((t1, s1, d1, e1), (t2, s2, d2, e2)):
            pltpu.sync_copy(sv.at[pl.ds(s * NCH, NCH)], srcv)
            pltpu.sync_copy(dv.at[pl.ds(s * NCH, NCH)], dstv)
            pltpu.sync_copy(ev.at[pl.ds(s * NCH, NCH)], eav)

            def chunk(i, carry):
                pltpu.async_copy(t.at[srcv.at[i]], rows, sem).wait()

                def scale(e, carry2):
                    a = eav[i, e]
                    for j in range(HALF // 16):
                        sl = pl.ds(j * 16, 16)
                        rows[e, sl] = rows[e, sl] * a
                    return carry2

                lax.fori_loop(0, B, scale, 0)
                pltpu.sync_copy(rows, acc.at[dstv.at[i]], add=True)
                return carry

            lax.fori_loop(0, NCH, chunk, 0)

        plsc.subcore_barrier()
        pltpu.sync_copy(acc.at[pl.ds(r0, RPT)],
                        out.at[pl.ds(r0, RPT), pl.ds(col0, HALF)])

    @pl.when(c == 0)
    def _half0():
        half(lna, t1a, t2a, 0)

    @pl.when(c == 1)
    def _half1():
        half(lnb, t1b, t2b, HALF)


_sc_block = functools.partial(
    pl.kernel,
    _sc_body,
    out_type=jax.ShapeDtypeStruct((N, H), jnp.float32),
    mesh=plsc.VectorSubcoreMesh(core_axis_name="c", subcore_axis_name="s"),
    scratch_types=[
        pltpu.VMEM_SHARED((N, HALF), jnp.float32),
        pltpu.VMEM((NCH, B), jnp.int32),
        pltpu.VMEM((NCH, B), jnp.int32),
        pltpu.VMEM((NCH, B), jnp.float32),
        pltpu.VMEM((B, HALF), jnp.float32),
        pltpu.SemaphoreType.DMA,
    ],
)()


def _block(x, wcat, s1, d1, e1, s2, d2, e2):
    lna, lnb, c1a, c1b, c2a, c2b = _matmul3(x, wcat)
    return _sc_block(lna, lnb, c1a, c1b, c2a, c2b, s1, d1, e1, s2, d2, e2)


def kernel(x, edge_index, edge_attr, edge_index2, edge_attr2, batch,
           W0_ln, W0_c1, W0_c2, W1_ln, W1_c1, W1_c2):
    s1 = edge_index[0].astype(jnp.int32).reshape(E // B, B)
    d1 = edge_index[1].astype(jnp.int32).reshape(E // B, B)
    e1 = edge_attr.reshape(E // B, B)
    s2 = edge_index2[0].astype(jnp.int32).reshape(E // B, B)
    d2 = edge_index2[1].astype(jnp.int32).reshape(E // B, B)
    e2 = edge_attr2.reshape(E // B, B)

    wcat0 = jnp.concatenate([W0_ln, W0_c1, W0_c2], axis=1)
    wcat1 = jnp.concatenate([W1_ln, W1_c1, W1_c2], axis=1)

    h = _block(x, wcat0, s1, d1, e1, s2, d2, e2)
    return _block(h, wcat1, s1, d1, e1, s2, d2, e2)


# speedup vs baseline: 3.0277x; 3.0277x over previous
import functools

import jax
import jax.numpy as jnp
from jax import lax
from jax.experimental import pallas as pl
from jax.experimental.pallas import tpu as pltpu
from jax.experimental.pallas import tpu_sc as plsc

N = 10000
D = 256
H = 256
E = 160000
HALF = 128

NS = 16
B = 128
NCH = 79
EPAD = NS * NCH * B
RPT = 624
REM = N - NS * RPT
MM_R = 1000


def _mm_body(x_ref, w_ref, lna, lnb, c1a, c1b, c2a, c2b):
    y = jnp.dot(x_ref[...], w_ref[...], preferred_element_type=jnp.float32)
    lna[...] = y[:, 0:128]
    lnb[...] = y[:, 128:256]
    c1a[...] = y[:, 256:384]
    c1b[...] = y[:, 384:512]
    c2a[...] = y[:, 512:640]
    c2b[...] = y[:, 640:768]


def _matmul3(x, wcat):
    return pl.pallas_call(
        _mm_body,
        grid=(N // MM_R,),
        in_specs=[
            pl.BlockSpec((MM_R, D), lambda i: (i, 0)),
            pl.BlockSpec((D, 3 * H), lambda i: (0, 0)),
        ],
        out_specs=[pl.BlockSpec((MM_R, HALF), lambda i: (i, 0))] * 6,
        out_shape=[jax.ShapeDtypeStruct((N, HALF), jnp.float32)] * 6,
    )(x, wcat)


def _sc_body(lna, lnb, t1a, t1b, t2a, t2b,
             s1, d1, e1, s2, d2, e2,
             out, acc, srcv, dstv, eav, rows, sem):
    c = lax.axis_index("c")
    s = lax.axis_index("s")
    r0 = s * RPT

    def half(ln, t1, t2, col0):
        pltpu.sync_copy(ln.at[pl.ds(r0, RPT)], acc.at[pl.ds(r0, RPT)])

        @pl.when(s == NS - 1)
        def _init_rem():
            pltpu.sync_copy(ln.at[pl.ds(NS * RPT, REM)],
                            acc.at[pl.ds(NS * RPT, REM)])

        plsc.subcore_barrier()

        for (t, sv, dv, ev) in ((t1, s1, d1, e1), (t2, s2, d2, e2)):
            pltpu.sync_copy(sv.at[s], srcv)
            pltpu.sync_copy(dv.at[s], dstv)
            pltpu.sync_copy(ev.at[s], eav)

            def chunk(i, carry):
                pltpu.async_copy(t.at[srcv.at[i]], rows, sem).wait()

                def scale(g, carry2):
                    ea16 = eav[i, pl.ds(g * 16, 16)]
                    for l in range(16):
                        a = ea16[l]
                        e = g * 16 + l
                        for j in range(HALF // 16):
                            sl = pl.ds(j * 16, 16)
                            rows[e, sl] = rows[e, sl] * a
                    return carry2

                lax.fori_loop(0, B // 16, scale, 0)
                pltpu.sync_copy(rows, acc.at[dstv.at[i]], add=True)
                return carry

            lax.fori_loop(0, NCH, chunk, 0)

        plsc.subcore_barrier()
        pltpu.sync_copy(acc.at[pl.ds(r0, RPT)],
                        out.at[pl.ds(r0, RPT), pl.ds(col0, HALF)])

        @pl.when(s == NS - 1)
        def _out_rem():
            pltpu.sync_copy(acc.at[pl.ds(NS * RPT, REM)],
                            out.at[pl.ds(NS * RPT, REM), pl.ds(col0, HALF)])

    @pl.when(c == 0)
    def _half0():
        half(lna, t1a, t2a, 0)

    @pl.when(c == 1)
    def _half1():
        half(lnb, t1b, t2b, HALF)


@functools.cache
def _sc_block():
    return pl.kernel(
        _sc_body,
        out_type=jax.ShapeDtypeStruct((N, H), jnp.float32),
        mesh=plsc.VectorSubcoreMesh(core_axis_name="c", subcore_axis_name="s"),
        scratch_types=[
            pltpu.VMEM_SHARED((N, HALF), jnp.float32),
            pltpu.VMEM((NCH, B), jnp.int32),
            pltpu.VMEM((NCH, B), jnp.int32),
            pltpu.VMEM((NCH, B), jnp.float32),
            pltpu.VMEM((B, HALF), jnp.float32),
            pltpu.SemaphoreType.DMA,
        ],
    )


def _block(x, wcat, s1, d1, e1, s2, d2, e2):
    lna, lnb, c1a, c1b, c2a, c2b = _matmul3(x, wcat)
    return _sc_block()(lna, lnb, c1a, c1b, c2a, c2b, s1, d1, e1, s2, d2, e2)


def kernel(x, edge_index, edge_attr, edge_index2, edge_attr2, batch,
           W0_ln, W0_c1, W0_c2, W1_ln, W1_c1, W1_c2):
    def _pad_i(v):
        return jnp.concatenate(
            [v.astype(jnp.int32), jnp.zeros((EPAD - E,), jnp.int32)]
        ).reshape(NS, NCH, B)

    def _pad_f(v):
        return jnp.concatenate(
            [v, jnp.zeros((EPAD - E,), jnp.float32)]
        ).reshape(NS, NCH, B)

    s1 = _pad_i(edge_index[0])
    d1 = _pad_i(edge_index[1])
    e1 = _pad_f(edge_attr)
    s2 = _pad_i(edge_index2[0])
    d2 = _pad_i(edge_index2[1])
    e2 = _pad_f(edge_attr2)

    wcat0 = jnp.concatenate([W0_ln, W0_c1, W0_c2], axis=1)
    wcat1 = jnp.concatenate([W1_ln, W1_c1, W1_c2], axis=1)

    h = _block(x, wcat0, s1, d1, e1, s2, d2, e2)
    return _block(h, wcat1, s1, d1, e1, s2, d2, e2)
